# baseline (speedup 1.0000x reference)
import jax
import jax.numpy as jnp
from jax.experimental import pallas as pl
from jax.experimental.pallas import tpu as pltpu

_BM = 400
_H = _BM // 2


def _fused_gcn_kernel(ca_ref, cb_ref, x_ref, w_ref, b_ref, o_ref):
    xv = x_ref[...]
    wv = w_ref[...]
    bv = b_ref[...][None, :]
    for h, c_ref in enumerate((ca_ref, cb_ref)):
        prop = jnp.dot(c_ref[...], xv, preferred_element_type=jnp.float32)
        lin = jax.lax.dot_general(
            prop, wv, (((1,), (1,)), ((), ())),
            preferred_element_type=jnp.float32,
        )
        o_ref[h * _H:(h + 1) * _H, :] = lin + bv


def kernel(x, edge_index, C, W, b):
    del edge_index
    n, d_in = x.shape
    d_out = W.shape[0]

    z = lambda i: i * 0
    return pl.pallas_call(
        _fused_gcn_kernel,
        grid=(n // _BM,),
        in_specs=[
            pl.BlockSpec((_H, n), lambda i: (2 * i, z(i))),
            pl.BlockSpec((_H, n), lambda i: (2 * i + 1, z(i))),
            pl.BlockSpec((n, d_in), lambda i: (z(i), z(i))),
            pl.BlockSpec((d_out, d_in), lambda i: (z(i), z(i))),
            pl.BlockSpec((d_out,), lambda i: (z(i),)),
        ],
        out_specs=pl.BlockSpec((_BM, d_out), lambda i: (i, z(i))),
        out_shape=jax.ShapeDtypeStruct((n, d_out), jnp.float32),
        compiler_params=pltpu.CompilerParams(
            vmem_limit_bytes=64 * 1024 * 1024,
            dimension_semantics=("parallel",),
        ),
    )(C, C, x, W, b)


# speedup vs baseline: 1.0881x; 1.0881x over previous
import jax
import jax.numpy as jnp
from jax.experimental import pallas as pl
from jax.experimental.pallas import tpu as pltpu

_BM = 400


def _fused_gcn_kernel(c_ref, x_ref, w_ref, b_ref, o_ref):
    prop = jnp.dot(c_ref[...], x_ref[...], preferred_element_type=jnp.float32)
    lin = jax.lax.dot_general(
        prop, w_ref[...], (((1,), (1,)), ((), ())),
        preferred_element_type=jnp.float32,
    )
    o_ref[...] = lin + b_ref[...][None, :]


def kernel(x, edge_index, C, W, b):
    del edge_index
    n, d_in = x.shape
    d_out = W.shape[0]

    z = lambda i: i * 0
    return pl.pallas_call(
        _fused_gcn_kernel,
        grid=(pl.cdiv(n, _BM),),
        in_specs=[
            pl.BlockSpec((_BM, n), lambda i: (i, z(i))),
            pl.BlockSpec((n, d_in), lambda i: (z(i), z(i))),
            pl.BlockSpec((d_out, d_in), lambda i: (z(i), z(i))),
            pl.BlockSpec((d_out,), lambda i: (z(i),)),
        ],
        out_specs=pl.BlockSpec((_BM, d_out), lambda i: (i, z(i))),
        out_shape=jax.ShapeDtypeStruct((n, d_out), jnp.float32),
        compiler_params=pltpu.CompilerParams(
            vmem_limit_bytes=64 * 1024 * 1024,
            dimension_semantics=("parallel",),
        ),
    )(C, x, W, b)
